# baseline (speedup 1.0000x reference)
import functools

import jax
import jax.numpy as jnp
from jax import lax
from jax.experimental import pallas as pl
from jax.experimental.pallas import tpu as pltpu
from jax.experimental.pallas import tpu_sc as plsc

MIN_INDEX = -10

NC = 2
NS = 16
NW = NC * NS
LANES = 16

CHUNK = 80


def _fuse_body(tables_ref, w1_ref, out_ref):
    for p in range(tables_ref.shape[0]):
        out_ref[p] = lax.dot_general(
            tables_ref[p], w1_ref[p],
            (((1,), (0,)), ((), ())),
            preferred_element_type=jnp.float32,
        )


def _fuse_tables(tables, w1r):
    P, NV, E = tables.shape
    return pl.pallas_call(
        _fuse_body,
        out_shape=jax.ShapeDtypeStruct((P, NV, E), jnp.float32),
    )(tables, w1r)


def _build_body(f_ref, out_ref):
    NV = f_ref.shape[1]
    i = pl.program_id(0)

    @pl.when(i < NV)
    def _triple():
        pair12 = f_ref[1][:, None, :] + f_ref[2][None, :, :]
        t = f_ref[0, pl.ds(jnp.minimum(i, NV - 1), 1)][0][None, None, :] + pair12
        out_ref[0] = t.reshape(NV * NV, -1)

    @pl.when(i == NV)
    def _pair34():
        out_ref[0] = (f_ref[3][:, None, :]
                      + f_ref[4][None, :, :]).reshape(NV * NV, -1)

    @pl.when(i == NV + 1)
    def _pair56():
        out_ref[0] = (f_ref[5][:, None, :]
                      + f_ref[6][None, :, :]).reshape(NV * NV, -1)


def _build_tables(F):
    P, NV, E = F.shape
    n_slabs = NV + 2
    out = pl.pallas_call(
        _build_body,
        grid=(n_slabs,),
        in_specs=[pl.BlockSpec((P, NV, E), lambda i: (0, 0, 0))],
        out_specs=pl.BlockSpec((1, NV * NV, E), lambda i: (i, 0, 0)),
        out_shape=jax.ShapeDtypeStruct((n_slabs, NV * NV, E), jnp.float32),
    )(F)
    return out.reshape(n_slabs * NV * NV, E)


def _make_gather_sum(N, G, E):
    tok_per_w = N // NW
    n_chunks = tok_per_w // CHUNK
    mesh = plsc.VectorSubcoreMesh(
        core_axis_name="c", subcore_axis_name="s",
        num_cores=NC, num_subcores=NS,
    )

    assert n_chunks % 2 == 0 and n_chunks >= 4
    half = n_chunks // 2

    scratch = (
        [pltpu.VMEM((G * CHUNK,), jnp.int32) for _ in range(2)]
        + [pltpu.VMEM((G * CHUNK, E), jnp.float32) for _ in range(2)]
        + [pltpu.VMEM((CHUNK, E), jnp.float32) for _ in range(2)]
        + [pltpu.SemaphoreType.DMA for _ in range(6)]
    )

    @functools.partial(
        pl.kernel,
        out_type=jax.ShapeDtypeStruct((N, E), jnp.float32),
        mesh=mesh,
        compiler_params=pltpu.CompilerParams(needs_layout_passes=False),
        scratch_types=scratch,
    )
    def gather_sum(f_hbm, rows_hbm, out_hbm,
                   idx0, idx1, rows0, rows1, o0, o1, *sems):
        idx_v = (idx0, idx1)
        rows_v = (rows0, rows1)
        o_v = (o0, o1)
        sem_i = sems[0:2]
        sem_g = sems[2:4]
        sem_o = sems[4:6]
        wid = lax.axis_index("s") * NC + lax.axis_index("c")
        base = wid * tok_per_w

        def idx_copy(b, c, wait):
            cid = wid * n_chunks + c
            cp = pltpu.make_async_copy(
                rows_hbm.at[pl.ds(cid * G * CHUNK, G * CHUNK)],
                idx_v[b], sem_i[b])
            cp.wait() if wait else cp.start()

        def gather(b, wait):
            cp = pltpu.make_async_copy(
                f_hbm.at[idx_v[b]], rows_v[b], sem_g[b])
            cp.wait() if wait else cp.start()

        def out_copy(b, c, wait):
            cp = pltpu.make_async_copy(
                o_v[b], out_hbm.at[pl.ds(base + c * CHUNK, CHUNK), :],
                sem_o[b])
            cp.wait() if wait else cp.start()

        def compute(b):
            @plsc.parallel_loop(0, CHUNK, unroll=2)
            def tok_body(t):
                for cc in range(E // LANES):
                    sl = pl.ds(cc * LANES, LANES)
                    vals = [rows_v[b][g * CHUNK + t, sl] for g in range(G)]
                    while len(vals) > 1:
                        vals = [a + b2 for a, b2 in zip(vals[::2], vals[1::2])] \
                            + ([vals[-1]] if len(vals) % 2 else [])
                    o_v[b][t, sl] = vals[0]

        idx_copy(0, 0, False)
        idx_copy(1, 1, False)
        idx_copy(0, 0, True)
        gather(0, False)

        def pair_body(c2, carry):
            for b in (0, 1):
                c = 2 * c2 + b
                gather(b, True)

                @pl.when(c2 < half - 1)
                def _prefetch_idx():
                    idx_copy(b, c + 2, False)

                b1 = 1 - b
                if b == 0:
                    idx_copy(b1, c + 1, True)
                    gather(b1, False)
                else:
                    @pl.when(c2 < half - 1)
                    def _next_gather():
                        idx_copy(b1, c + 1, True)
                        gather(b1, False)

                @pl.when(c2 >= 1)
                def _drain_out():
                    out_copy(b, c - 2, True)

                compute(b)
                out_copy(b, c, False)
            return carry

        lax.fori_loop(0, half, pair_body, 0, unroll=False)
        out_copy(0, n_chunks - 2, True)
        out_copy(1, n_chunks - 1, True)

    return gather_sum


def _tail_body(S, g_ref, b1_ref, w2_ref, b2_ref, out_ref):
    h = jnp.maximum(g_ref[...] + b1_ref[...], 0.0)
    res = lax.dot_general(
        h, w2_ref[...],
        (((1,), (0,)), ((), ())),
        preferred_element_type=jnp.float32,
    ) + b2_ref[...]
    out_ref[...] = res.reshape(out_ref.shape)


def _mlp_tail(g, b1, w2, b2, B, S, bb):
    N, E = g.shape
    grid = (B // bb,)
    return pl.pallas_call(
        functools.partial(_tail_body, S),
        grid=grid,
        in_specs=[
            pl.BlockSpec((bb * S, E), lambda i: (i, 0)),
            pl.BlockSpec((1, E), lambda i: (0, 0)),
            pl.BlockSpec((E, E), lambda i: (0, 0)),
            pl.BlockSpec((1, E), lambda i: (0, 0)),
        ],
        out_specs=pl.BlockSpec((bb, S, E), lambda i: (i, 0, 0)),
        out_shape=jax.ShapeDtypeStruct((B, S, E), jnp.float32),
    )(g, b1, w2, b2)


@jax.jit
def kernel(integral, tables, W1, b1, W2, b2):
    P, NV, E = tables.shape
    orig_shape = integral.shape[:-1]
    N = 1
    for d in orig_shape:
        N *= d

    F = _fuse_tables(tables, W1.reshape(P, E, E))
    table = _build_tables(F)

    idx = jnp.clip(integral.reshape(N, P).astype(jnp.int32) - MIN_INDEX, 0, NV - 1)
    r0 = (idx[:, 0] * NV + idx[:, 1]) * NV + idx[:, 2]
    r1 = idx[:, 3] * NV + idx[:, 4] + NV * NV * NV
    r2 = idx[:, 5] * NV + idx[:, 6] + NV * NV * NV + NV * NV
    rows = jnp.stack([r0, r1, r2], axis=1)
    G = 3
    rows_c = rows.T.reshape(G, N // CHUNK, CHUNK).transpose(1, 0, 2).reshape(-1)

    g = _make_gather_sum(N, G, E)(table, rows_c)

    B, S = orig_shape
    return _mlp_tail(g, b1.reshape(1, E), W2, b2.reshape(1, E), B, S, bb=16)
